# Initial kernel scaffold; baseline (speedup 1.0000x reference)
import jax
import jax.numpy as jnp
from jax.experimental import pallas as pl


def kernel(positions, species, senders, receivers, embed, Wr1_0, Wr2_0, Wmix_0, Wself_0, Wout_0, Wr1_1, Wr2_1, Wmix_1, Wself_1, Wout_1):
    raise NotImplementedError("write your pallas kernel here")


# speedup vs baseline: 1.6205x; 1.6205x over previous
import functools
import math

import jax
import jax.numpy as jnp
from jax import lax
from jax.experimental import pallas as pl
from jax.experimental.pallas import tpu as pltpu
from jax.experimental.pallas import tpu_sc as plsc

_N = 10000
_E = 160000
_C = 128
_BESS = 8
_S = 9
_R_MAX = 5.0
_AVG_NEIGH = 16.0
_OUT = 128

_NC = 2
_NS = 16
_NW = _NC * _NS
_EPW = _E // _NW
_CH = 128
_NFULL = _EPW // _CH
_TAIL = _EPW - _NFULL * _CH
_NPT = 624
_NREM = _N - _NS * _NPT

_mesh = plsc.VectorSubcoreMesh(core_axis_name="c", subcore_axis_name="s")


@functools.partial(
    pl.kernel, mesh=_mesh,
    out_type=jax.ShapeDtypeStruct((_E, _C), jnp.float32),
    scratch_types=[
        pltpu.VMEM((_CH,), jnp.int32),
        pltpu.VMEM((_CH, _C), jnp.float32),
        pltpu.VMEM((_TAIL,), jnp.int32),
        pltpu.VMEM((_TAIL, _C), jnp.float32),
        pltpu.SemaphoreType.DMA,
    ])
def _gather_h(h_hbm, snd_hbm, out_hbm, idx_v, rows_v, idx_t, rows_t, sem):
    wid = lax.axis_index("s") * _NC + lax.axis_index("c")
    base = wid * _EPW

    def body(j, carry):
        off = base + j * _CH
        pltpu.sync_copy(snd_hbm.at[pl.ds(off, _CH)], idx_v)
        pltpu.async_copy(h_hbm.at[idx_v], rows_v, sem).wait()
        pltpu.sync_copy(rows_v, out_hbm.at[pl.ds(off, _CH)])
        return carry

    lax.fori_loop(0, _NFULL, body, 0)
    off = base + _NFULL * _CH
    pltpu.sync_copy(snd_hbm.at[pl.ds(off, _TAIL)], idx_t)
    pltpu.async_copy(h_hbm.at[idx_t], rows_t, sem).wait()
    pltpu.sync_copy(rows_t, out_hbm.at[pl.ds(off, _TAIL)])


@functools.partial(
    pl.kernel, mesh=_mesh,
    out_type=(jax.ShapeDtypeStruct((_N, _C), jnp.float32),
              jax.ShapeDtypeStruct((_N, _C), jnp.float32)),
    scratch_types=[
        pltpu.VMEM((_CH,), jnp.int32),
        pltpu.VMEM((_CH, _C), jnp.float32),
        pltpu.VMEM((_TAIL,), jnp.int32),
        pltpu.VMEM((_TAIL, _C), jnp.float32),
        pltpu.VMEM_SHARED((_N, _C), jnp.float32),
        pltpu.SemaphoreType.DMA,
    ])
def _scatter_add(z_hbm, rcv_hbm, zeros_hbm, a0_hbm, a1_hbm,
                 idx_v, rows_v, idx_t, rows_t, acc, sem):
    cid = lax.axis_index("c")
    sid = lax.axis_index("s")

    myrows = pl.ds(sid * _NPT, _NPT)
    tailrows = pl.ds(_NS * _NPT, _NREM)
    pltpu.sync_copy(zeros_hbm.at[myrows], acc.at[myrows])

    @pl.when(sid == 0)
    def _():
        pltpu.sync_copy(zeros_hbm.at[tailrows], acc.at[tailrows])

    plsc.subcore_barrier()

    base = (sid * _NC + cid) * _EPW

    def body(j, carry):
        off = base + j * _CH
        pltpu.sync_copy(rcv_hbm.at[pl.ds(off, _CH)], idx_v)
        pltpu.sync_copy(z_hbm.at[pl.ds(off, _CH)], rows_v)
        pltpu.sync_copy(rows_v, acc.at[idx_v], add=True)
        return carry

    lax.fori_loop(0, _NFULL, body, 0)
    off = base + _NFULL * _CH
    pltpu.sync_copy(rcv_hbm.at[pl.ds(off, _TAIL)], idx_t)
    pltpu.sync_copy(z_hbm.at[pl.ds(off, _TAIL)], rows_t)
    pltpu.sync_copy(rows_t, acc.at[idx_t], add=True)
    plsc.subcore_barrier()

    @pl.when(cid == 0)
    def _():
        pltpu.sync_copy(acc.at[myrows], a0_hbm.at[myrows])

        @pl.when(sid == 0)
        def _():
            pltpu.sync_copy(acc.at[tailrows], a0_hbm.at[tailrows])

    @pl.when(cid == 1)
    def _():
        pltpu.sync_copy(acc.at[myrows], a1_hbm.at[myrows])

        @pl.when(sid == 0)
        def _():
            pltpu.sync_copy(acc.at[tailrows], a1_hbm.at[tailrows])


_NBE = 1000


def _embed_body(sp_ref, emb_ref, h_ref):
    sp = sp_ref[0, 0, :]
    oh = (sp[:, None] == lax.broadcasted_iota(jnp.int32, (_NBE, 16), 1))
    h_ref[...] = jnp.dot(oh.astype(jnp.float32), emb_ref[...],
                         preferred_element_type=jnp.float32)


_embed_call = pl.pallas_call(
    _embed_body,
    grid=(_N // _NBE,),
    in_specs=[
        pl.BlockSpec((1, 1, _NBE), lambda i: (i, 0, 0)),
        pl.BlockSpec((16, _C), lambda i: (0, 0)),
    ],
    out_specs=pl.BlockSpec((_NBE, _C), lambda i: (i, 0)),
    out_shape=jax.ShapeDtypeStruct((_N, _C), jnp.float32),
)

_EB = 2000
_SQ2R = math.sqrt(2.0 / _R_MAX)
_C0 = 0.28209479177387814
_C1 = 0.4886025119029199
_C2A = 1.0925484305920792
_C2B = 0.31539156525252005
_C2C = 0.5462742152960396


def _edge_body(ps_ref, pr_ref, hs_ref, wr1_ref, wr2_ref, w2p_ref, z_ref):
    ps = ps_ref[...]
    pr = pr_ref[...]
    vx = pr[:, 0:1] - ps[:, 0:1]
    vy = pr[:, 1:2] - ps[:, 1:2]
    vz = pr[:, 2:3] - ps[:, 2:3]
    d = jnp.sqrt(vx * vx + vy * vy + vz * vz)
    inv = 1.0 / (d + 1e-6)
    ux = vx * inv
    uy = vy * inv
    uz = vz * inv
    u = d * (1.0 / _R_MAX)
    env = jnp.where(u < 1.0, (1.0 - u) * (1.0 - u) * (1.0 + 2.0 * u), 0.0)
    scale = env * inv * _SQ2R
    rb = jnp.concatenate([jnp.sin((math.pi * n) * u) for n in range(1, _BESS + 1)],
                         axis=1) * scale
    r1 = jnp.maximum(jnp.dot(rb, wr1_ref[...], preferred_element_type=jnp.float32), 0.0)
    rr = jnp.dot(r1, wr2_ref[...], preferred_element_type=jnp.float32)
    msg = hs_ref[...] * rr
    ys = (_C0 * jnp.ones_like(u), _C1 * uy, _C1 * uz, _C1 * ux,
          _C2A * ux * uy, _C2A * uy * uz, _C2B * (3.0 * uz * uz - 1.0),
          _C2A * ux * uz, _C2C * (ux * ux - uy * uy))
    g = jnp.concatenate([(msg * y).astype(jnp.bfloat16) for y in ys], axis=1)
    z_ref[...] = jnp.dot(g, w2p_ref[...], preferred_element_type=jnp.float32)


_edge_call = pl.pallas_call(
    _edge_body,
    grid=(_E // _EB,),
    in_specs=[
        pl.BlockSpec((_EB, _C), lambda i: (i, 0)),
        pl.BlockSpec((_EB, _C), lambda i: (i, 0)),
        pl.BlockSpec((_EB, _C), lambda i: (i, 0)),
        pl.BlockSpec((_BESS, 64), lambda i: (0, 0)),
        pl.BlockSpec((64, _C), lambda i: (0, 0)),
        pl.BlockSpec((_S * _C, _C), lambda i: (0, 0)),
    ],
    out_specs=pl.BlockSpec((_EB, _C), lambda i: (i, 0)),
    out_shape=jax.ShapeDtypeStruct((_E, _C), jnp.float32),
)

_NB = 2000


def _node_body(a0_ref, a1_ref, h_ref, wself_ref, wout_ref, hn_ref, out_ref):
    pre = (a0_ref[...] + a1_ref[...]) * (1.0 / _AVG_NEIGH)
    pre = pre + jnp.dot(h_ref[...], wself_ref[...], preferred_element_type=jnp.float32)
    hn = jnp.tanh(pre)
    hn_ref[...] = hn
    out_ref[...] = jnp.dot(hn, wout_ref[...], preferred_element_type=jnp.float32)


_node_call = pl.pallas_call(
    _node_body,
    grid=(_N // _NB,),
    in_specs=[
        pl.BlockSpec((_NB, _C), lambda i: (i, 0)),
        pl.BlockSpec((_NB, _C), lambda i: (i, 0)),
        pl.BlockSpec((_NB, _C), lambda i: (i, 0)),
        pl.BlockSpec((_C, _C), lambda i: (0, 0)),
        pl.BlockSpec((_C, _OUT), lambda i: (0, 0)),
    ],
    out_specs=[
        pl.BlockSpec((_NB, _C), lambda i: (i, 0)),
        pl.BlockSpec((_NB, _OUT), lambda i: (i, 0)),
    ],
    out_shape=[
        jax.ShapeDtypeStruct((_N, _C), jnp.float32),
        jax.ShapeDtypeStruct((_N, _OUT), jnp.float32),
    ],
)


def kernel(positions, species, senders, receivers, embed,
           Wr1_0, Wr2_0, Wmix_0, Wself_0, Wout_0,
           Wr1_1, Wr2_1, Wmix_1, Wself_1, Wout_1):
    senders = senders.astype(jnp.int32)
    receivers = receivers.astype(jnp.int32)
    pos128 = jnp.pad(positions, ((0, 0), (0, _C - 3)))
    emb16 = jnp.pad(embed, ((0, 16 - embed.shape[0]), (0, 0)))
    sp3 = species.astype(jnp.int32).reshape(_N // _NBE, 1, _NBE)

    def reorder(w):
        return (w.reshape(_C, _S, _C).transpose(1, 0, 2)
                 .reshape(_S * _C, _C).astype(jnp.bfloat16))

    layers = [(Wr1_0, Wr2_0, reorder(Wmix_0), Wself_0, Wout_0),
              (Wr1_1, Wr2_1, reorder(Wmix_1), Wself_1, Wout_1)]

    zeros_nc = jnp.zeros((_N, _C), jnp.float32)
    ps = _gather_h(pos128, senders)
    pr = _gather_h(pos128, receivers)
    h = _embed_call(sp3, emb16)
    outs = []
    for (wr1, wr2, w2p, wself, wout) in layers:
        hs = _gather_h(h, senders)
        z = _edge_call(ps, pr, hs, wr1, wr2, w2p)
        a0, a1 = _scatter_add(z, receivers, zeros_nc)
        h, out = _node_call(a0, a1, h, wself, wout)
        outs.append(out)
    return jnp.concatenate(outs, axis=1)
